# baseline (speedup 1.0000x reference)
import functools

import jax
import jax.numpy as jnp
from jax import lax
from jax.experimental import pallas as pl
from jax.experimental.pallas import tpu as pltpu
from jax.experimental.pallas import tpu_sc as plsc

NUM_EMB = 1002
EMBED = 128
B, L = 4096, 200
N = B * L
NC, NS = 2, 16
NW = NC * NS
PER_W = N // NW
GATHER = 64
NI = PER_W // GATHER
CHUNK = GATHER
NJ = PER_W // CHUNK
NBUF = 8
AHEAD = NBUF // 2
assert (NJ - 2 * AHEAD) % NBUF == 0 and NBUF == 2 * AHEAD


def _emb_body(idx_hbm, w_hbm, out_hbm, w_sh, idx_v, rows_v, gsem, osem):
    cid = lax.axis_index("c")
    sid = lax.axis_index("s")
    wid = sid * NC + cid
    base = wid * PER_W

    @pl.when(sid == 0)
    def _():
        pltpu.sync_copy(w_hbm, w_sh)

    pltpu.sync_copy(idx_hbm.at[wid], idx_v)
    plsc.subcore_barrier()

    def gather(j, buf):
        pltpu.async_copy(w_sh.at[idx_v.at[j]], rows_v.at[buf], gsem)

    def put(j, buf):
        pltpu.async_copy(
            rows_v.at[buf], out_hbm.at[pl.ds(base + j * CHUNK, CHUNK)], osem
        )

    def wait_gather(buf):
        pltpu.make_async_copy(w_sh.at[idx_v.at[0]], rows_v.at[buf], gsem).wait()

    def wait_put(j, buf):
        pltpu.make_async_copy(
            rows_v.at[buf], out_hbm.at[pl.ds(base + j * CHUNK, CHUNK)], osem
        ).wait()

    for j in range(AHEAD):
        gather(j, j % NBUF)
    for j in range(AHEAD):
        gather(j + AHEAD, (j + AHEAD) % NBUF)
        wait_gather(j % NBUF)
        put(j, j % NBUF)

    @pl.loop(AHEAD, NJ - AHEAD, step=NBUF)
    def _steady(j0):
        for b in range(NBUF):
            j = j0 + b
            buf = (AHEAD + b) % NBUF
            nbuf = (buf + AHEAD) % NBUF
            wait_put(j - AHEAD, nbuf)
            gather(j + AHEAD, nbuf)
            wait_gather(buf)
            put(j, buf)

    for j in range(NJ - AHEAD, NJ):
        wait_put(j - AHEAD, (j - AHEAD) % NBUF)
        wait_gather(j % NBUF)
        put(j, j % NBUF)
    for j in range(NJ - AHEAD, NJ):
        wait_put(j, j % NBUF)


@functools.partial(jax.jit, static_argnames=())
def kernel(indices, W):
    idx = indices.reshape(NW, NI, GATHER)
    mesh = plsc.VectorSubcoreMesh(
        core_axis_name="c", subcore_axis_name="s", num_cores=NC, num_subcores=NS
    )
    run = pl.kernel(
        _emb_body,
        out_type=jax.ShapeDtypeStruct((N, EMBED), jnp.float32),
        mesh=mesh,
        scratch_types=[
            pltpu.VMEM_SHARED((NUM_EMB, EMBED), jnp.float32),
            pltpu.VMEM((NI, GATHER), jnp.int32),
            pltpu.VMEM((NBUF, CHUNK, EMBED), jnp.float32),
            pltpu.SemaphoreType.DMA,
            pltpu.SemaphoreType.DMA,
        ],
    )
    out = run(idx, W)
    return out.reshape(B, L, EMBED)


# speedup vs baseline: 1.0032x; 1.0032x over previous
import functools

import jax
import jax.numpy as jnp
from jax import lax
from jax.experimental import pallas as pl
from jax.experimental.pallas import tpu as pltpu
from jax.experimental.pallas import tpu_sc as plsc

NUM_EMB = 1002
EMBED = 128
B, L = 4096, 200
N = B * L
NC, NS = 2, 16
NW = NC * NS
PER_W = N // NW
GATHER = 128
NI = PER_W // GATHER
CHUNK = GATHER
NJ = PER_W // CHUNK
NBUF = 4
AHEAD = NBUF // 2
assert (NJ - 2 * AHEAD) % NBUF == 0 and NBUF == 2 * AHEAD


def _emb_body(idx_hbm, w_hbm, out_hbm, w_sh, idx_v, rows_v, gsem, osem):
    cid = lax.axis_index("c")
    sid = lax.axis_index("s")
    wid = sid * NC + cid
    base = wid * PER_W

    @pl.when(sid == 0)
    def _():
        pltpu.sync_copy(w_hbm, w_sh)

    pltpu.sync_copy(idx_hbm.at[wid], idx_v)
    plsc.subcore_barrier()

    def gather(j, buf):
        pltpu.async_copy(w_sh.at[idx_v.at[j]], rows_v.at[buf], gsem)

    def put(j, buf):
        pltpu.async_copy(
            rows_v.at[buf], out_hbm.at[pl.ds(base + j * CHUNK, CHUNK)], osem
        )

    def wait_gather(buf):
        pltpu.make_async_copy(w_sh.at[idx_v.at[0]], rows_v.at[buf], gsem).wait()

    def wait_put(j, buf):
        pltpu.make_async_copy(
            rows_v.at[buf], out_hbm.at[pl.ds(base + j * CHUNK, CHUNK)], osem
        ).wait()

    for j in range(AHEAD):
        gather(j, j % NBUF)
    for j in range(AHEAD):
        gather(j + AHEAD, (j + AHEAD) % NBUF)
        wait_gather(j % NBUF)
        put(j, j % NBUF)

    @pl.loop(AHEAD, NJ - AHEAD, step=NBUF)
    def _steady(j0):
        for b in range(NBUF):
            j = j0 + b
            buf = (AHEAD + b) % NBUF
            nbuf = (buf + AHEAD) % NBUF
            wait_put(j - AHEAD, nbuf)
            gather(j + AHEAD, nbuf)
            wait_gather(buf)
            put(j, buf)

    for j in range(NJ - AHEAD, NJ):
        wait_put(j - AHEAD, (j - AHEAD) % NBUF)
        wait_gather(j % NBUF)
        put(j, j % NBUF)
    for j in range(NJ - AHEAD, NJ):
        wait_put(j, j % NBUF)


@functools.partial(jax.jit, static_argnames=())
def kernel(indices, W):
    idx = indices.reshape(NW, NI, GATHER)
    mesh = plsc.VectorSubcoreMesh(
        core_axis_name="c", subcore_axis_name="s", num_cores=NC, num_subcores=NS
    )
    run = pl.kernel(
        _emb_body,
        out_type=jax.ShapeDtypeStruct((N, EMBED), jnp.float32),
        mesh=mesh,
        scratch_types=[
            pltpu.VMEM_SHARED((NUM_EMB, EMBED), jnp.float32),
            pltpu.VMEM((NI, GATHER), jnp.int32),
            pltpu.VMEM((NBUF, CHUNK, EMBED), jnp.float32),
            pltpu.SemaphoreType.DMA,
            pltpu.SemaphoreType.DMA,
        ],
    )
    out = run(idx, W)
    return out.reshape(B, L, EMBED)
